# Initial kernel scaffold; baseline (speedup 1.0000x reference)
import jax
import jax.numpy as jnp
from jax.experimental import pallas as pl


def kernel(x, embedding):
    raise NotImplementedError("write your pallas kernel here")


# speedup vs baseline: 1.0579x; 1.0579x over previous
import functools

import jax
import jax.numpy as jnp
from jax import lax
from jax.experimental import pallas as pl
from jax.experimental.pallas import tpu as pltpu
from jax.experimental.pallas import tpu_sc as plsc

NUM_E = 1024
DIM = 64
ROWS_PER_BLOCK = 512


def _argmin_body(x_ref, xsq_ref, embt_ref, esq_ref, idx_ref, acc_ref):
    i = pl.program_id(0)
    s = lax.dot_general(
        x_ref[...], embt_ref[...], (((1,), (0,)), ((), ())),
        preferred_element_type=jnp.float32)
    d2 = (xsq_ref[...] - 2.0 * s) + esq_ref[...]
    dc = jnp.maximum(d2, 0.0)
    dist = jnp.where(dc == 0.0, 0.0, dc * lax.rsqrt(dc))
    m = jnp.min(dist, axis=1, keepdims=True)
    lane = lax.broadcasted_iota(jnp.int32, dist.shape, 1)
    idx_ref[...] = jnp.min(
        jnp.where(dist == m, lane, jnp.int32(NUM_E)), axis=1)
    part = jnp.sum(m * m)

    @pl.when(i == 0)
    def _init():
        acc_ref[...] = jnp.full_like(acc_ref, part)

    @pl.when(i > 0)
    def _accum():
        acc_ref[...] = acc_ref[...] + part


def _tc_argmin(flat_x, xsq, embt, esq):
    n = flat_x.shape[0]
    r = ROWS_PER_BLOCK
    grid = n // r
    return pl.pallas_call(
        _argmin_body,
        grid=(grid,),
        in_specs=[
            pl.BlockSpec((r, DIM), lambda i: (i, 0)),
            pl.BlockSpec((r, 1), lambda i: (i, 0)),
            pl.BlockSpec((DIM, NUM_E), lambda i: (0, 0)),
            pl.BlockSpec((1, NUM_E), lambda i: (0, 0)),
        ],
        out_specs=[
            pl.BlockSpec((r,), lambda i: (i,)),
            pl.BlockSpec((8, 128), lambda i: (0, 0)),
        ],
        out_shape=[
            jax.ShapeDtypeStruct((n,), jnp.int32),
            jax.ShapeDtypeStruct((8, 128), jnp.float32),
        ],
    )(flat_x, xsq, embt, esq)


def _sc_gather(embedding, idx):
    n = idx.shape[0]
    info = plsc.get_sparse_core_info()
    nc, ns = info.num_cores, info.num_subcores
    nw = nc * ns
    per_w = n // nw
    ch = 72
    k_ch = per_w // ch
    mesh = plsc.VectorSubcoreMesh(core_axis_name="c", subcore_axis_name="s")

    @functools.partial(
        pl.kernel,
        out_type=jax.ShapeDtypeStruct((n, DIM), jnp.float32),
        mesh=mesh,
        compiler_params=pltpu.CompilerParams(use_tc_tiling_on_sc=False),
        scratch_types=(
            [pltpu.VMEM((ch,), jnp.int32) for _ in range(k_ch)]
            + [pltpu.VMEM((ch, DIM), jnp.float32) for _ in range(k_ch)]
            + [pltpu.SemaphoreType.DMA]
        ),
    )
    def gather_k(table_hbm, idx_hbm, out_hbm, *scr):
        idx_bufs, row_bufs, sem = scr[:k_ch], scr[k_ch:2 * k_ch], scr[-1]
        wid = lax.axis_index("s") * nc + lax.axis_index("c")
        base = wid * per_w
        for c in range(k_ch):
            pltpu.sync_copy(idx_hbm.at[pl.ds(base + c * ch, ch)], idx_bufs[c])
        copies = [
            pltpu.async_copy(table_hbm.at[idx_bufs[c]], row_bufs[c], sem)
            for c in range(k_ch)
        ]
        for cp in copies:
            cp.wait()
        for c in range(k_ch):
            pltpu.sync_copy(row_bufs[c], out_hbm.at[pl.ds(base + c * ch, ch)])

    return gather_k(embedding, idx)


def kernel(x, embedding):
    b, l, d = x.shape
    flat_x = x.reshape(-1, d)
    xsq = jnp.sum(flat_x ** 2, axis=1, keepdims=True)
    esq = jnp.sum(embedding ** 2, axis=1)[None, :]
    embt = embedding.T
    idx, acc = _tc_argmin(flat_x, xsq, embt, esq)
    zq = _sc_gather(embedding, idx)
    loss = 1.25 * acc[0, 0] / (b * l * d)
    return (zq.reshape(b, l, d), loss)


# speedup vs baseline: 1.1090x; 1.0483x over previous
import functools

import jax
import jax.numpy as jnp
from jax import lax
from jax.experimental import pallas as pl
from jax.experimental.pallas import tpu as pltpu
from jax.experimental.pallas import tpu_sc as plsc

NUM_E = 1024
DIM = 64
ROWS_PER_BLOCK = 2048


def _argmin_body(x_ref, xsq_ref, emb_ref, esq_ref, idx_ref, acc_ref):
    i = pl.program_id(0)
    s = lax.dot_general(
        x_ref[...], emb_ref[...], (((1,), (1,)), ((), ())),
        preferred_element_type=jnp.float32)
    d2 = (xsq_ref[...] - 2.0 * s) + esq_ref[...]
    dc = jnp.maximum(d2, 0.0)
    dist = jnp.where(dc == 0.0, 0.0, dc * lax.rsqrt(dc))
    m = jnp.min(dist, axis=1, keepdims=True)
    lane = lax.broadcasted_iota(jnp.int32, dist.shape, 1)
    idx_ref[...] = jnp.min(
        jnp.where(dist == m, lane, jnp.int32(NUM_E)), axis=1)
    part = jnp.sum(m * m)

    @pl.when(i == 0)
    def _init():
        acc_ref[...] = jnp.full_like(acc_ref, part)

    @pl.when(i > 0)
    def _accum():
        acc_ref[...] = acc_ref[...] + part


def _tc_argmin(flat_x, xsq, emb, esq):
    n = flat_x.shape[0]
    r = ROWS_PER_BLOCK
    grid = n // r
    return pl.pallas_call(
        _argmin_body,
        grid=(grid,),
        in_specs=[
            pl.BlockSpec((r, DIM), lambda i: (i, 0)),
            pl.BlockSpec((r, 1), lambda i: (i, 0)),
            pl.BlockSpec((NUM_E, DIM), lambda i: (0, 0)),
            pl.BlockSpec((1, NUM_E), lambda i: (0, 0)),
        ],
        out_specs=[
            pl.BlockSpec((r,), lambda i: (i,)),
            pl.BlockSpec((8, 128), lambda i: (0, 0)),
        ],
        out_shape=[
            jax.ShapeDtypeStruct((n,), jnp.int32),
            jax.ShapeDtypeStruct((8, 128), jnp.float32),
        ],
    )(flat_x, xsq, emb, esq)


def _sc_gather(embedding, idx):
    n = idx.shape[0]
    info = plsc.get_sparse_core_info()
    nc, ns = info.num_cores, info.num_subcores
    nw = nc * ns
    per_w = n // nw
    ch = 96
    k_ch = per_w // ch
    mesh = plsc.VectorSubcoreMesh(core_axis_name="c", subcore_axis_name="s")

    @functools.partial(
        pl.kernel,
        out_type=jax.ShapeDtypeStruct((n, DIM), jnp.float32),
        mesh=mesh,
        compiler_params=pltpu.CompilerParams(use_tc_tiling_on_sc=False),
        scratch_types=(
            [pltpu.VMEM((ch,), jnp.int32) for _ in range(k_ch)]
            + [pltpu.VMEM((ch, DIM), jnp.float32) for _ in range(k_ch)]
            + [pltpu.SemaphoreType.DMA]
        ),
    )
    def gather_k(table_hbm, idx_hbm, out_hbm, *scr):
        idx_bufs, row_bufs, sem = scr[:k_ch], scr[k_ch:2 * k_ch], scr[-1]
        wid = lax.axis_index("s") * nc + lax.axis_index("c")
        base = wid * per_w
        for c in range(k_ch):
            pltpu.sync_copy(idx_hbm.at[pl.ds(base + c * ch, ch)], idx_bufs[c])
        copies = [
            pltpu.async_copy(table_hbm.at[idx_bufs[c]], row_bufs[c], sem)
            for c in range(k_ch)
        ]
        for cp in copies:
            cp.wait()
        for c in range(k_ch):
            pltpu.sync_copy(row_bufs[c], out_hbm.at[pl.ds(base + c * ch, ch)])

    return gather_k(embedding, idx)


def kernel(x, embedding):
    b, l, d = x.shape
    flat_x = x.reshape(-1, d)
    xsq = jnp.sum(flat_x ** 2, axis=1, keepdims=True)
    esq = jnp.sum(embedding ** 2, axis=1)[None, :]
    idx, acc = _tc_argmin(flat_x, xsq, embedding, esq)
    zq = _sc_gather(embedding, idx)
    loss = 1.25 * acc[0, 0] / (b * l * d)
    return (zq.reshape(b, l, d), loss)


# speedup vs baseline: 1.3673x; 1.2329x over previous
import functools

import jax
import jax.numpy as jnp
from jax import lax
from jax.experimental import pallas as pl
from jax.experimental.pallas import tpu as pltpu
from jax.experimental.pallas import tpu_sc as plsc

NUM_E = 1024
DIM = 64
BATCH_PER_BLOCK = 8


def _argmin_body(xt_ref, xsq_ref, embt_ref, esq_ref, idx_ref, acc_ref):
    i = pl.program_id(0)
    part = None
    for b in range(BATCH_PER_BLOCK):
        s = lax.dot_general(
            embt_ref[...], xt_ref[b], (((0,), (0,)), ((), ())),
            preferred_element_type=jnp.float32)
        d2 = (xsq_ref[pl.ds(b, 1), :] - 2.0 * s) + esq_ref[...]
        dc = jnp.maximum(d2, 0.0)
        dist = jnp.where(dc == 0.0, 0.0, dc * lax.rsqrt(dc))
        m = jnp.min(dist, axis=0, keepdims=True)
        code = lax.broadcasted_iota(jnp.int32, dist.shape, 0)
        idx_ref[pl.ds(b, 1), :] = jnp.min(
            jnp.where(dist == m, code, jnp.int32(NUM_E)), axis=0, keepdims=True)
        p = jnp.sum(m * m)
        part = p if part is None else part + p

    @pl.when(i == 0)
    def _init():
        acc_ref[...] = jnp.full_like(acc_ref, part)

    @pl.when(i > 0)
    def _accum():
        acc_ref[...] = acc_ref[...] + part


def _tc_argmin(xt, xsq, embt, esq):
    nb, _, length = xt.shape
    r = BATCH_PER_BLOCK
    grid = nb // r
    return pl.pallas_call(
        _argmin_body,
        grid=(grid,),
        in_specs=[
            pl.BlockSpec((r, DIM, length), lambda i: (i, 0, 0)),
            pl.BlockSpec((r, length), lambda i: (i, 0)),
            pl.BlockSpec((DIM, NUM_E), lambda i: (0, 0)),
            pl.BlockSpec((NUM_E, 1), lambda i: (0, 0)),
        ],
        out_specs=[
            pl.BlockSpec((r, length), lambda i: (i, 0)),
            pl.BlockSpec((8, 128), lambda i: (0, 0)),
        ],
        out_shape=[
            jax.ShapeDtypeStruct((nb, length), jnp.int32),
            jax.ShapeDtypeStruct((8, 128), jnp.float32),
        ],
    )(xt, xsq, embt, esq)


def _sc_gather(embedding, idx):
    n = idx.shape[0]
    info = plsc.get_sparse_core_info()
    nc, ns = info.num_cores, info.num_subcores
    nw = nc * ns
    per_w = n // nw
    ch = 96
    k_ch = per_w // ch
    mesh = plsc.VectorSubcoreMesh(core_axis_name="c", subcore_axis_name="s")

    @functools.partial(
        pl.kernel,
        out_type=jax.ShapeDtypeStruct((n, DIM), jnp.float32),
        mesh=mesh,
        compiler_params=pltpu.CompilerParams(use_tc_tiling_on_sc=False),
        scratch_types=(
            [pltpu.VMEM((ch,), jnp.int32) for _ in range(k_ch)]
            + [pltpu.VMEM((ch, DIM), jnp.float32) for _ in range(k_ch)]
            + [pltpu.SemaphoreType.DMA]
        ),
    )
    def gather_k(table_hbm, idx_hbm, out_hbm, *scr):
        idx_bufs, row_bufs, sem = scr[:k_ch], scr[k_ch:2 * k_ch], scr[-1]
        wid = lax.axis_index("s") * nc + lax.axis_index("c")
        base = wid * per_w
        for c in range(k_ch):
            pltpu.sync_copy(idx_hbm.at[pl.ds(base + c * ch, ch)], idx_bufs[c])
        copies = [
            pltpu.async_copy(table_hbm.at[idx_bufs[c]], row_bufs[c], sem)
            for c in range(k_ch)
        ]
        for cp in copies:
            cp.wait()
        for c in range(k_ch):
            pltpu.sync_copy(row_bufs[c], out_hbm.at[pl.ds(base + c * ch, ch)])

    return gather_k(embedding, idx)


def kernel(x, embedding):
    b, l, d = x.shape
    xsq = jnp.sum(x ** 2, axis=2)
    esq = jnp.sum(embedding ** 2, axis=1)[:, None]
    xt = jnp.transpose(x, (0, 2, 1))
    embt = embedding.T
    idx, acc = _tc_argmin(xt, xsq, embt, esq)
    zq = _sc_gather(embedding, idx.reshape(b * l))
    loss = 1.25 * acc[0, 0] / (b * l * d)
    return (zq.reshape(b, l, d), loss)


# speedup vs baseline: 1.4043x; 1.0270x over previous
import functools

import jax
import jax.numpy as jnp
from jax import lax
from jax.experimental import pallas as pl
from jax.experimental.pallas import tpu as pltpu
from jax.experimental.pallas import tpu_sc as plsc

NUM_E = 1024
DIM = 64
BATCH_PER_BLOCK = 8


def _argmin_body(xt_ref, xsq_ref, embt_ref, esq_ref, idx_ref, acc_ref):
    i = pl.program_id(0)
    length = xt_ref.shape[2]
    code = lax.broadcasted_iota(
        jnp.int32, (NUM_E, length), 0).astype(jnp.float32)
    part = None
    for b in range(BATCH_PER_BLOCK):
        s = lax.dot_general(
            embt_ref[...], xt_ref[b], (((0,), (0,)), ((), ())),
            preferred_element_type=jnp.float32)
        d2 = (xsq_ref[pl.ds(b, 1), :] - 2.0 * s) + esq_ref[...]
        dc = jnp.maximum(d2, 0.0)
        dist = jnp.where(dc == 0.0, 0.0, dc * lax.rsqrt(dc))
        m = jnp.min(dist, axis=0, keepdims=True)
        idxf = jnp.min(
            jnp.where(dist == m, code, jnp.float32(NUM_E)), axis=0,
            keepdims=True)
        idx_ref[pl.ds(b, 1), :] = idxf.astype(jnp.int32)
        p = jnp.sum(m * m)
        part = p if part is None else part + p

    @pl.when(i == 0)
    def _init():
        acc_ref[...] = jnp.full_like(acc_ref, part)

    @pl.when(i > 0)
    def _accum():
        acc_ref[...] = acc_ref[...] + part


def _tc_argmin(xt, xsq, embt, esq):
    nb, _, length = xt.shape
    r = BATCH_PER_BLOCK
    grid = nb // r
    return pl.pallas_call(
        _argmin_body,
        grid=(grid,),
        in_specs=[
            pl.BlockSpec((r, DIM, length), lambda i: (i, 0, 0)),
            pl.BlockSpec((r, length), lambda i: (i, 0)),
            pl.BlockSpec((DIM, NUM_E), lambda i: (0, 0)),
            pl.BlockSpec((NUM_E, 1), lambda i: (0, 0)),
        ],
        out_specs=[
            pl.BlockSpec((r, length), lambda i: (i, 0)),
            pl.BlockSpec((8, 128), lambda i: (0, 0)),
        ],
        out_shape=[
            jax.ShapeDtypeStruct((nb, length), jnp.int32),
            jax.ShapeDtypeStruct((8, 128), jnp.float32),
        ],
    )(xt, xsq, embt, esq)


def _sc_gather(embedding, idx):
    n = idx.shape[0]
    info = plsc.get_sparse_core_info()
    nc, ns = info.num_cores, info.num_subcores
    nw = nc * ns
    per_w = n // nw
    ch = 96
    k_ch = per_w // ch
    mesh = plsc.VectorSubcoreMesh(core_axis_name="c", subcore_axis_name="s")

    @functools.partial(
        pl.kernel,
        out_type=jax.ShapeDtypeStruct((n, DIM), jnp.float32),
        mesh=mesh,
        compiler_params=pltpu.CompilerParams(use_tc_tiling_on_sc=False),
        scratch_types=(
            [pltpu.VMEM((ch,), jnp.int32) for _ in range(k_ch)]
            + [pltpu.VMEM((ch, DIM), jnp.float32) for _ in range(k_ch)]
            + [pltpu.SemaphoreType.DMA, pltpu.SemaphoreType.DMA]
        ),
    )
    def gather_k(table_hbm, idx_hbm, out_hbm, *scr):
        idx_bufs, row_bufs = scr[:k_ch], scr[k_ch:2 * k_ch]
        sem, sem_out = scr[-2], scr[-1]
        wid = lax.axis_index("s") * nc + lax.axis_index("c")
        base = wid * per_w
        for c in range(k_ch):
            pltpu.sync_copy(idx_hbm.at[pl.ds(base + c * ch, ch)], idx_bufs[c])
        copies = [
            pltpu.async_copy(table_hbm.at[idx_bufs[c]], row_bufs[c], sem)
            for c in range(k_ch)
        ]
        outs = []
        for c in range(k_ch):
            copies[c].wait()
            outs.append(pltpu.async_copy(
                row_bufs[c], out_hbm.at[pl.ds(base + c * ch, ch)], sem_out))
        for cp in outs:
            cp.wait()

    return gather_k(embedding, idx)


def kernel(x, embedding):
    b, l, d = x.shape
    xsq = jnp.sum(x ** 2, axis=2)
    esq = jnp.sum(embedding ** 2, axis=1)[:, None]
    xt = jnp.transpose(x, (0, 2, 1))
    embt = embedding.T
    idx, acc = _tc_argmin(xt, xsq, embt, esq)
    zq = _sc_gather(embedding, idx.reshape(b * l))
    loss = 1.25 * acc[0, 0] / (b * l * d)
    return (zq.reshape(b, l, d), loss)


# speedup vs baseline: 1.5573x; 1.1090x over previous
import functools

import jax
import jax.numpy as jnp
from jax import lax
from jax.experimental import pallas as pl
from jax.experimental.pallas import tpu as pltpu
from jax.experimental.pallas import tpu_sc as plsc

NUM_E = 1024
DIM = 64
BATCH_PER_BLOCK = 8


def _argmin_body(xt_ref, xsq_ref, embt_ref, esq_ref, idx_ref, acc_ref):
    i = pl.program_id(0)
    length = xt_ref.shape[2]
    code = lax.broadcasted_iota(
        jnp.int32, (NUM_E, length), 0).astype(jnp.float32)
    part = None
    for b in range(BATCH_PER_BLOCK):
        s = lax.dot_general(
            embt_ref[...], xt_ref[b], (((0,), (0,)), ((), ())),
            preferred_element_type=jnp.float32)
        d2 = (xsq_ref[pl.ds(b, 1), :] - 2.0 * s) + esq_ref[...]
        dc = jnp.maximum(d2, 0.0)
        dist = jnp.where(dc == 0.0, 0.0, dc * lax.rsqrt(dc))
        m = jnp.min(dist, axis=0, keepdims=True)
        idxf = jnp.min(
            jnp.where(dist == m, code, jnp.float32(NUM_E)), axis=0,
            keepdims=True)
        idx_ref[pl.ds(b, 1), :] = idxf.astype(jnp.int32)
        p = jnp.sum(m * m)
        part = p if part is None else part + p

    @pl.when(i == 0)
    def _init():
        acc_ref[...] = jnp.full_like(acc_ref, part)

    @pl.when(i > 0)
    def _accum():
        acc_ref[...] = acc_ref[...] + part


def _tc_argmin(xt, xsq, embt, esq):
    nb, _, length = xt.shape
    r = BATCH_PER_BLOCK
    grid = nb // r
    return pl.pallas_call(
        _argmin_body,
        grid=(grid,),
        in_specs=[
            pl.BlockSpec((r, DIM, length), lambda i: (i, 0, 0)),
            pl.BlockSpec((r, length), lambda i: (i, 0)),
            pl.BlockSpec((DIM, NUM_E), lambda i: (0, 0)),
            pl.BlockSpec((NUM_E, 1), lambda i: (0, 0)),
        ],
        out_specs=[
            pl.BlockSpec((r, length), lambda i: (i, 0)),
            pl.BlockSpec((8, 128), lambda i: (0, 0)),
        ],
        out_shape=[
            jax.ShapeDtypeStruct((nb, length), jnp.int32),
            jax.ShapeDtypeStruct((8, 128), jnp.float32),
        ],
    )(xt, xsq, embt, esq)


def _sc_gather(embedding, idx):
    n = idx.shape[0]
    info = plsc.get_sparse_core_info()
    nc, ns = info.num_cores, info.num_subcores
    nw = nc * ns
    per_w = n // nw
    ch = 96
    k_ch = per_w // ch
    mesh = plsc.VectorSubcoreMesh(core_axis_name="c", subcore_axis_name="s")

    @functools.partial(
        pl.kernel,
        out_type=jax.ShapeDtypeStruct((n, 2 * DIM), jnp.float32),
        mesh=mesh,
        compiler_params=pltpu.CompilerParams(use_tc_tiling_on_sc=False),
        scratch_types=(
            [pltpu.VMEM((per_w,), jnp.int32)]
            + [pltpu.VMEM((ch, DIM), jnp.float32) for _ in range(k_ch)]
            + [pltpu.SemaphoreType.DMA, pltpu.SemaphoreType.DMA]
        ),
    )
    def gather_k(table_hbm, idx_hbm, out_hbm, *scr):
        idx_all, row_bufs = scr[0], scr[1:1 + k_ch]
        sem, sem_out = scr[-2], scr[-1]
        wid = lax.axis_index("s") * nc + lax.axis_index("c")
        base = wid * per_w
        pltpu.sync_copy(idx_hbm.at[pl.ds(base, per_w)], idx_all)
        copies = [
            pltpu.async_copy(
                table_hbm.at[idx_all.at[pl.ds(c * ch, ch)]], row_bufs[c], sem)
            for c in range(k_ch)
        ]
        outs = []
        for c in range(k_ch):
            copies[c].wait()
            outs.append(pltpu.async_copy(
                row_bufs[c],
                out_hbm.at[pl.ds(base + c * ch, ch), pl.ds(0, DIM)], sem_out))
        for cp in outs:
            cp.wait()

    return gather_k(embedding, idx)


def kernel(x, embedding):
    b, l, d = x.shape
    xsq = jnp.sum(x ** 2, axis=2)
    esq = jnp.sum(embedding ** 2, axis=1)[:, None]
    xt = jnp.transpose(x, (0, 2, 1))
    embt = embedding.T
    idx, acc = _tc_argmin(xt, xsq, embt, esq)
    zq_pad = _sc_gather(embedding, idx.reshape(b * l))
    loss = 1.25 * acc[0, 0] / (b * l * d)
    return (zq_pad[:, :d].reshape(b, l, d), loss)


# speedup vs baseline: 1.5873x; 1.0193x over previous
import functools

import jax
import jax.numpy as jnp
from jax import lax
from jax.experimental import pallas as pl
from jax.experimental.pallas import tpu as pltpu
from jax.experimental.pallas import tpu_sc as plsc

NUM_E = 1024
DIM = 64
BATCH_PER_BLOCK = 16


def _argmin_body(xt_ref, xsq_ref, embt_ref, esq_ref, idx_ref, acc_ref):
    i = pl.program_id(0)
    length = xt_ref.shape[2]
    code = lax.broadcasted_iota(
        jnp.int32, (NUM_E, length), 0).astype(jnp.float32)
    part = None
    for b in range(BATCH_PER_BLOCK):
        s = lax.dot_general(
            embt_ref[...], xt_ref[b], (((0,), (0,)), ((), ())),
            preferred_element_type=jnp.float32)
        d2 = (xsq_ref[pl.ds(b, 1), :] - 2.0 * s) + esq_ref[...]
        dc = jnp.maximum(d2, 0.0)
        dist = jnp.where(dc == 0.0, 0.0, dc * lax.rsqrt(dc))
        m = jnp.min(dist, axis=0, keepdims=True)
        idxf = jnp.min(
            jnp.where(dist == m, code, jnp.float32(NUM_E)), axis=0,
            keepdims=True)
        idx_ref[pl.ds(b, 1), :] = idxf.astype(jnp.int32)
        p = jnp.sum(m * m)
        part = p if part is None else part + p

    @pl.when(i == 0)
    def _init():
        acc_ref[...] = jnp.full_like(acc_ref, part)

    @pl.when(i > 0)
    def _accum():
        acc_ref[...] = acc_ref[...] + part


def _tc_argmin(xt, xsq, embt, esq):
    nb, _, length = xt.shape
    r = BATCH_PER_BLOCK
    grid = nb // r
    return pl.pallas_call(
        _argmin_body,
        grid=(grid,),
        in_specs=[
            pl.BlockSpec((r, DIM, length), lambda i: (i, 0, 0)),
            pl.BlockSpec((r, length), lambda i: (i, 0)),
            pl.BlockSpec((DIM, NUM_E), lambda i: (0, 0)),
            pl.BlockSpec((NUM_E, 1), lambda i: (0, 0)),
        ],
        out_specs=[
            pl.BlockSpec((r, length), lambda i: (i, 0)),
            pl.BlockSpec((8, 128), lambda i: (0, 0)),
        ],
        out_shape=[
            jax.ShapeDtypeStruct((nb, length), jnp.int32),
            jax.ShapeDtypeStruct((8, 128), jnp.float32),
        ],
    )(xt, xsq, embt, esq)


def _sc_gather(embedding, idx):
    n = idx.shape[0]
    info = plsc.get_sparse_core_info()
    nc, ns = info.num_cores, info.num_subcores
    nw = nc * ns
    per_w = n // nw
    ch = 96
    k_ch = per_w // ch
    mesh = plsc.VectorSubcoreMesh(core_axis_name="c", subcore_axis_name="s")

    @functools.partial(
        pl.kernel,
        out_type=jax.ShapeDtypeStruct((n, 2 * DIM), jnp.float32),
        mesh=mesh,
        compiler_params=pltpu.CompilerParams(use_tc_tiling_on_sc=False),
        scratch_types=(
            [pltpu.VMEM((per_w,), jnp.int32)]
            + [pltpu.VMEM((ch, DIM), jnp.float32) for _ in range(k_ch)]
            + [pltpu.SemaphoreType.DMA, pltpu.SemaphoreType.DMA]
        ),
    )
    def gather_k(table_hbm, idx_hbm, out_hbm, *scr):
        idx_all, row_bufs = scr[0], scr[1:1 + k_ch]
        sem, sem_out = scr[-2], scr[-1]
        wid = lax.axis_index("s") * nc + lax.axis_index("c")
        base = wid * per_w
        pltpu.sync_copy(idx_hbm.at[pl.ds(base, per_w)], idx_all)
        copies = [
            pltpu.async_copy(
                table_hbm.at[idx_all.at[pl.ds(c * ch, ch)]], row_bufs[c], sem)
            for c in range(k_ch)
        ]
        outs = []
        for c in range(k_ch):
            copies[c].wait()
            outs.append(pltpu.async_copy(
                row_bufs[c],
                out_hbm.at[pl.ds(base + c * ch, ch), pl.ds(0, DIM)], sem_out))
        for cp in outs:
            cp.wait()

    return gather_k(embedding, idx)


def kernel(x, embedding):
    b, l, d = x.shape
    xsq = jnp.sum(x ** 2, axis=2)
    esq = jnp.sum(embedding ** 2, axis=1)[:, None]
    xt = jnp.transpose(x, (0, 2, 1))
    embt = embedding.T
    idx, acc = _tc_argmin(xt, xsq, embt, esq)
    zq_pad = _sc_gather(embedding, idx.reshape(b * l))
    loss = 1.25 * acc[0, 0] / (b * l * d)
    return (zq_pad[:, :d].reshape(b, l, d), loss)
